# speedup vs baseline: 6.1177x; 6.1177x over previous
import jax
import jax.numpy as jnp
from jax import lax
from jax.experimental import pallas as pl

_DIM = 1024
_E = 64
_T = 32768
_EPS = 1e-06
_BT = 1024
_BT2 = 4096


def _pass1_body(x_ref, wt_ref, b_ref, masked_ref, colsum_ref):
    s = jnp.dot(x_ref[...], wt_ref[...], preferred_element_type=jnp.float32)
    s = s + b_ref[0:1, :]
    m = jnp.max(s, axis=1, keepdims=True)
    e = jnp.exp(s - m)
    z = jnp.sum(e, axis=1, keepdims=True)
    p = e / z
    pm = jnp.max(p, axis=1, keepdims=True)
    iota = lax.broadcasted_iota(jnp.int32, p.shape, 1)
    cand = jnp.where(p == pm, iota, _E)
    idx = jnp.min(cand, axis=1, keepdims=True)
    masked = jnp.where(iota == idx, p, 0.0)
    masked_ref[...] = masked
    part = jnp.sum(masked, axis=0, keepdims=True)

    @pl.when(pl.program_id(0) == 0)
    def _():
        colsum_ref[...] = jnp.zeros_like(colsum_ref)

    colsum_ref[...] += jnp.broadcast_to(part, colsum_ref.shape)


def _pass2_body(masked_ref, colsum_ref, out_ref):
    denom = colsum_ref[0:1, :] + _EPS
    out_ref[...] = masked_ref[...] / denom * float(_T)


def kernel(x, W, b):
    wt = W.T
    b2 = jnp.broadcast_to(b.reshape(1, _E), (8, _E))

    grid1 = (_T // _BT,)
    masked, colsum = pl.pallas_call(
        _pass1_body,
        grid=grid1,
        in_specs=[
            pl.BlockSpec((_BT, _DIM), lambda i: (i, 0)),
            pl.BlockSpec((_DIM, _E), lambda i: (0, 0)),
            pl.BlockSpec((8, _E), lambda i: (0, 0)),
        ],
        out_specs=[
            pl.BlockSpec((_BT, _E), lambda i: (i, 0)),
            pl.BlockSpec((8, _E), lambda i: (0, 0)),
        ],
        out_shape=[
            jax.ShapeDtypeStruct((_T, _E), jnp.float32),
            jax.ShapeDtypeStruct((8, _E), jnp.float32),
        ],
    )(x, wt, b2)

    grid2 = (_T // _BT2,)
    out = pl.pallas_call(
        _pass2_body,
        grid=grid2,
        in_specs=[
            pl.BlockSpec((_BT2, _E), lambda i: (i, 0)),
            pl.BlockSpec((8, _E), lambda i: (0, 0)),
        ],
        out_specs=pl.BlockSpec((_BT2, _E), lambda i: (i, 0)),
        out_shape=jax.ShapeDtypeStruct((_T, _E), jnp.float32),
    )(masked, colsum)
    return out


# speedup vs baseline: 6.2588x; 1.0231x over previous
import jax
import jax.numpy as jnp
from jax import lax
from jax.experimental import pallas as pl

_DIM = 1024
_E = 64
_T = 32768
_EPS = 1e-06
_BT = 1024
_BT2 = 4096


def _pass1_body(x_ref, wt_ref, b_ref, masked_ref, colsum_ref):
    s = jnp.dot(x_ref[...], wt_ref[...], preferred_element_type=jnp.float32)
    s = s + b_ref[0:1, :]
    m = jnp.max(s, axis=1, keepdims=True)
    e = jnp.exp(s - m)
    z = jnp.sum(e, axis=1, keepdims=True)
    v = 1.0 / z
    iota = lax.broadcasted_iota(jnp.int32, s.shape, 1)
    cand = jnp.where(s == m, iota, _E)
    idx = jnp.min(cand, axis=1, keepdims=True)
    masked = jnp.where(iota == idx, v, 0.0)
    masked_ref[...] = masked
    part = jnp.sum(masked, axis=0, keepdims=True)

    @pl.when(pl.program_id(0) == 0)
    def _():
        colsum_ref[...] = jnp.zeros_like(colsum_ref)

    colsum_ref[...] += jnp.broadcast_to(part, colsum_ref.shape)


def _pass2_body(masked_ref, colsum_ref, out_ref):
    denom = colsum_ref[0:1, :] + _EPS
    out_ref[...] = masked_ref[...] / denom * float(_T)


def kernel(x, W, b):
    wt = W.T
    b2 = jnp.broadcast_to(b.reshape(1, _E), (8, _E))

    grid1 = (_T // _BT,)
    masked, colsum = pl.pallas_call(
        _pass1_body,
        grid=grid1,
        in_specs=[
            pl.BlockSpec((_BT, _DIM), lambda i: (i, 0)),
            pl.BlockSpec((_DIM, _E), lambda i: (0, 0)),
            pl.BlockSpec((8, _E), lambda i: (0, 0)),
        ],
        out_specs=[
            pl.BlockSpec((_BT, _E), lambda i: (i, 0)),
            pl.BlockSpec((8, _E), lambda i: (0, 0)),
        ],
        out_shape=[
            jax.ShapeDtypeStruct((_T, _E), jnp.float32),
            jax.ShapeDtypeStruct((8, _E), jnp.float32),
        ],
    )(x, wt, b2)

    grid2 = (_T // _BT2,)
    out = pl.pallas_call(
        _pass2_body,
        grid=grid2,
        in_specs=[
            pl.BlockSpec((_BT2, _E), lambda i: (i, 0)),
            pl.BlockSpec((8, _E), lambda i: (0, 0)),
        ],
        out_specs=pl.BlockSpec((_BT2, _E), lambda i: (i, 0)),
        out_shape=jax.ShapeDtypeStruct((_T, _E), jnp.float32),
    )(masked, colsum)
    return out


# speedup vs baseline: 7.1347x; 1.1399x over previous
import jax
import jax.numpy as jnp
from jax import lax
from jax.experimental import pallas as pl
from jax.experimental.pallas import tpu as pltpu

_DIM = 1024
_E = 64
_T = 32768
_EPS = 1e-06
_BT = 1024


def _body(x_ref, wt_ref, b_ref, out_ref, colsum_ref):
    j = pl.program_id(0)
    s = jnp.dot(x_ref[...], wt_ref[...], preferred_element_type=jnp.float32)
    s = s + b_ref[0:1, :]
    m = jnp.max(s, axis=1, keepdims=True)
    e = jnp.exp(s - m)
    z = jnp.sum(e, axis=1, keepdims=True)
    v = 1.0 / z
    iota = lax.broadcasted_iota(jnp.int32, s.shape, 1)
    cand = jnp.where(s == m, iota, _E)
    idx = jnp.min(cand, axis=1, keepdims=True)
    masked = jnp.where(iota == idx, v, 0.0)
    out_ref[pl.ds(j * _BT, _BT), :] = masked
    part = jnp.sum(masked, axis=0, keepdims=True)

    @pl.when(j == 0)
    def _():
        colsum_ref[...] = jnp.zeros_like(colsum_ref)

    colsum_ref[...] += jnp.broadcast_to(part, colsum_ref.shape)

    @pl.when(j == pl.num_programs(0) - 1)
    def _():
        denom = colsum_ref[0:1, :] + _EPS
        out_ref[...] = out_ref[...] / denom * float(_T)


def kernel(x, W, b):
    wt = W.T
    b2 = jnp.broadcast_to(b.reshape(1, _E), (8, _E))

    out = pl.pallas_call(
        _body,
        grid=(_T // _BT,),
        in_specs=[
            pl.BlockSpec((_BT, _DIM), lambda i: (i, 0)),
            pl.BlockSpec((_DIM, _E), lambda i: (0, 0)),
            pl.BlockSpec((8, _E), lambda i: (0, 0)),
        ],
        out_specs=pl.BlockSpec((_T, _E), lambda i: (0, 0)),
        out_shape=jax.ShapeDtypeStruct((_T, _E), jnp.float32),
        scratch_shapes=[pltpu.VMEM((8, _E), jnp.float32)],
    )(x, wt, b2)
    return out


# speedup vs baseline: 8.1187x; 1.1379x over previous
import jax
import jax.numpy as jnp
from jax import lax
from jax.experimental import pallas as pl
from jax.experimental.pallas import tpu as pltpu

_DIM = 1024
_E = 64
_T = 32768
_EPS = 1e-06
_BT = 2048


def _body(x_ref, wt_ref, b_ref, out_ref, colsum_ref):
    j = pl.program_id(0)
    s = jnp.dot(x_ref[...], wt_ref[...], preferred_element_type=jnp.float32)
    s = s + b_ref[0:1, :]
    m = jnp.max(s, axis=1, keepdims=True)
    e = jnp.exp(s - m)
    z = jnp.sum(e, axis=1, keepdims=True)
    v = 1.0 / z
    iota = lax.broadcasted_iota(jnp.int32, s.shape, 1)
    cand = jnp.where(s == m, iota, _E)
    idx = jnp.min(cand, axis=1, keepdims=True)
    masked = jnp.where(iota == idx, v, 0.0)
    out_ref[pl.ds(j * _BT, _BT), :] = masked
    part = jnp.sum(masked, axis=0, keepdims=True)

    @pl.when(j == 0)
    def _():
        colsum_ref[...] = jnp.zeros_like(colsum_ref)

    colsum_ref[...] += jnp.broadcast_to(part, colsum_ref.shape)

    @pl.when(j == pl.num_programs(0) - 1)
    def _():
        denom = colsum_ref[0:1, :] + _EPS
        out_ref[...] = out_ref[...] / denom * float(_T)


def kernel(x, W, b):
    wt = W.T
    b2 = jnp.broadcast_to(b.reshape(1, _E), (8, _E))

    out = pl.pallas_call(
        _body,
        grid=(_T // _BT,),
        in_specs=[
            pl.BlockSpec((_BT, _DIM), lambda i: (i, 0)),
            pl.BlockSpec((_DIM, _E), lambda i: (0, 0)),
            pl.BlockSpec((8, _E), lambda i: (0, 0)),
        ],
        out_specs=pl.BlockSpec((_T, _E), lambda i: (0, 0)),
        out_shape=jax.ShapeDtypeStruct((_T, _E), jnp.float32),
        scratch_shapes=[pltpu.VMEM((8, _E), jnp.float32)],
    )(x, wt, b2)
    return out


# speedup vs baseline: 8.4200x; 1.0371x over previous
import jax
import jax.numpy as jnp
from jax import lax
from jax.experimental import pallas as pl
from jax.experimental.pallas import tpu as pltpu

_DIM = 1024
_E = 64
_T = 32768
_EPS = 1e-06
_BT = 4096


def _body(x_ref, wt_ref, b_ref, out_ref, colsum_ref):
    j = pl.program_id(0)
    s = jnp.dot(x_ref[...], wt_ref[...], preferred_element_type=jnp.float32)
    s = s + b_ref[0:1, :]
    m = jnp.max(s, axis=1, keepdims=True)
    e = jnp.exp(s - m)
    z = jnp.sum(e, axis=1, keepdims=True)
    v = 1.0 / z
    iota = lax.broadcasted_iota(jnp.int32, s.shape, 1)
    cand = jnp.where(s == m, iota, _E)
    idx = jnp.min(cand, axis=1, keepdims=True)
    masked = jnp.where(iota == idx, v, 0.0)
    out_ref[pl.ds(j * _BT, _BT), :] = masked
    part = jnp.sum(masked, axis=0, keepdims=True)

    @pl.when(j == 0)
    def _():
        colsum_ref[...] = jnp.zeros_like(colsum_ref)

    colsum_ref[...] += jnp.broadcast_to(part, colsum_ref.shape)

    @pl.when(j == pl.num_programs(0) - 1)
    def _():
        denom = colsum_ref[0:1, :] + _EPS
        out_ref[...] = out_ref[...] / denom * float(_T)


def kernel(x, W, b):
    wt = W.T
    b2 = jnp.broadcast_to(b.reshape(1, _E), (8, _E))

    out = pl.pallas_call(
        _body,
        grid=(_T // _BT,),
        in_specs=[
            pl.BlockSpec((_BT, _DIM), lambda i: (i, 0)),
            pl.BlockSpec((_DIM, _E), lambda i: (0, 0)),
            pl.BlockSpec((8, _E), lambda i: (0, 0)),
        ],
        out_specs=pl.BlockSpec((_T, _E), lambda i: (0, 0)),
        out_shape=jax.ShapeDtypeStruct((_T, _E), jnp.float32),
        scratch_shapes=[pltpu.VMEM((8, _E), jnp.float32)],
    )(x, wt, b2)
    return out
